# baseline (speedup 1.0000x reference)
import functools

import jax
import jax.numpy as jnp
from jax import lax
from jax.experimental import pallas as pl
from jax.experimental.pallas import tpu as pltpu
from jax.experimental.pallas import tpu_sc as plsc

EMB = 32
BATCH = 16384
NC = 2
NS = 16
NW = NC * NS
BPW = BATCH // NW
CHUNK = 128
NCH = BPW // CHUNK

BM = 2048


def _gather_kernel(uidx_hbm, midx_hbm, utab_hbm, mtab_hbm, uout_hbm, mout_hbm,
                   uidx_v, midx_v, rows_v, sem):
    wid = lax.axis_index("s") * NC + lax.axis_index("c")
    base = wid * BPW
    pltpu.sync_copy(uidx_hbm.at[pl.ds(wid * NCH, NCH)], uidx_v)
    pltpu.sync_copy(midx_hbm.at[pl.ds(wid * NCH, NCH)], midx_v)
    seq = [(tab, idx_v, j, out)
           for (tab, idx_v, out) in ((utab_hbm, uidx_v, uout_hbm),
                                     (mtab_hbm, midx_v, mout_hbm))
           for j in range(NCH)]
    pend = [None, None]

    def flush(t):
        d, pout, pj = pend[t % 2]
        d.wait()
        pltpu.sync_copy(rows_v.at[t % 2],
                        pout.at[pl.ds(base + pj * CHUNK, CHUNK)])

    for t, (tab, idx_v, j, out) in enumerate(seq):
        pend[t % 2] = (pltpu.async_copy(
            tab.at[idx_v.at[j]], rows_v.at[t % 2], sem), out, j)
        if t >= 1:
            flush(t - 1)
    flush(len(seq) - 1)


def _gather(uidx, midx, utab, mtab):
    mesh = plsc.VectorSubcoreMesh(core_axis_name="c", subcore_axis_name="s")
    k = functools.partial(
        pl.kernel,
        mesh=mesh,
        out_type=[
            jax.ShapeDtypeStruct((BATCH, EMB), jnp.float32),
            jax.ShapeDtypeStruct((BATCH, EMB), jnp.float32),
        ],
        scratch_types=[
            pltpu.VMEM((NCH, CHUNK), jnp.int32),
            pltpu.VMEM((NCH, CHUNK), jnp.int32),
            pltpu.VMEM((2, CHUNK, EMB), jnp.float32),
            pltpu.SemaphoreType.DMA,
        ],
        compiler_params=pltpu.CompilerParams(use_tc_tiling_on_sc=False),
    )(_gather_kernel)
    return k(uidx.reshape(NW * NCH, CHUNK), midx.reshape(NW * NCH, CHUNK),
             utab, mtab)


def _mlp_kernel(u_any, m_any, w1u_ref, w1m_ref, b1_ref,
                w2_ref, b2_ref, w3t_ref, b3_ref, out_ref, u_v, m_v, sem):
    i = pl.program_id(0)
    cu = pltpu.make_async_copy(u_any.at[pl.ds(i * BM, BM)], u_v, sem)
    cu.start()
    cm = pltpu.make_async_copy(m_any.at[pl.ds(i * BM, BM)], m_v, sem)
    cm.start()
    cu.wait()
    cm.wait()
    h = jnp.dot(u_v[...], w1u_ref[...], preferred_element_type=jnp.float32)
    h = h + jnp.dot(m_v[...], w1m_ref[...],
                    preferred_element_type=jnp.float32)
    h = jnp.maximum(h + b1_ref[...], 0.0)
    h = jnp.dot(h, w2_ref[...], preferred_element_type=jnp.float32)
    h = jnp.maximum(h + b2_ref[...], 0.0)
    o = jnp.sum(h * w3t_ref[...], axis=1, keepdims=True) + b3_ref[...]
    out_ref[...] = 1.0 / (1.0 + jnp.exp(-o))


def _mlp(u, m, W1, b1, W2, b2, W3, b3):
    hid = W1.shape[1]
    h2 = W2.shape[1]
    grid = (BATCH // BM,)
    full = lambda shape: pl.BlockSpec(shape, lambda i: (0, 0))
    anyspec = pl.BlockSpec(memory_space=pltpu.HBM)
    out = pl.pallas_call(
        _mlp_kernel,
        grid=grid,
        in_specs=[
            anyspec,
            anyspec,
            full((EMB, hid)),
            full((EMB, hid)),
            full((1, hid)),
            full((hid, h2)),
            full((1, h2)),
            full((1, h2)),
            full((1, 1)),
        ],
        out_specs=pl.BlockSpec((BM, 1), lambda i: (i, 0)),
        out_shape=jax.ShapeDtypeStruct((BATCH, 1), jnp.float32),
        scratch_shapes=[
            pltpu.VMEM((BM, EMB), jnp.float32),
            pltpu.VMEM((BM, EMB), jnp.float32),
            pltpu.SemaphoreType.DMA,
        ],
    )(u, m, W1[:EMB], W1[EMB:], b1.reshape(1, hid), W2,
      b2.reshape(1, h2), W3.reshape(1, h2), b3.reshape(1, 1))
    return out


def kernel(user, movie, user_emb_table, movie_emb_table, W1, b1, W2, b2, W3, b3):
    user = user.astype(jnp.int32)
    movie = movie.astype(jnp.int32)
    u, m = _gather(user, movie, user_emb_table, movie_emb_table)
    out = _mlp(u, m, W1, b1, W2, b2, W3, b3)
    return jnp.squeeze(out, axis=-1)


# speedup vs baseline: 1.4569x; 1.4569x over previous
import functools

import jax
import jax.numpy as jnp
from jax import lax
from jax.experimental import pallas as pl
from jax.experimental.pallas import tpu as pltpu
from jax.experimental.pallas import tpu_sc as plsc

EMB = 32
BLK = 8
BATCH = 16384
NC = 2
NS = 16
NW = NC * NS
BPW = BATCH // NW
GS = 16
NGRP = 4
NG = BPW // GS

BM = 2048


def _gather_kernel(idx_hbm, tab_hbm, out_hbm, idx_v, blk_v, res_v,
                   sem0, sem1, sem2, sem3):
    wid = lax.axis_index("s") * NC + lax.axis_index("c")
    w8 = pl.multiple_of(wid * 8, 8)
    pltpu.sync_copy(idx_hbm.at[pl.ds(w8, 8)], idx_v)
    lane = lax.iota(jnp.int32, 16)
    sems = [sem0, sem1, sem2, sem3]

    def idx_vec(g):
        return idx_v[g // 8, pl.ds(jnp.remainder(g, 8) * 16, 16)]

    def issue(g, base_slot, sem):
        vec = idx_vec(g)
        for j in range(GS):
            e16 = jnp.max(jnp.where(lane == j, vec, 0))
            b = pl.multiple_of(jnp.bitwise_and(e16, -BLK), BLK)
            pltpu.async_copy(tab_hbm.at[pl.ds(b, BLK)],
                             blk_v.at[base_slot + j], sem)

    def process(g, base_slot, sem):
        for j in range(GS):
            pltpu.make_async_copy(tab_hbm.at[pl.ds(0, BLK)],
                                  blk_v.at[base_slot + j], sem).wait()
        vec = idx_vec(g)
        sub = jnp.bitwise_and(vec, BLK - 1)
        for j in range(GS):
            sv = jnp.full((16,), jnp.max(jnp.where(lane == j, sub, 0)),
                          jnp.int32)
            rv = jnp.full((16,), g * 4 + j // 4, jnp.int32)
            c = (j % 4) * EMB
            v0 = plsc.load_gather(blk_v.at[base_slot + j], [sv, lane])
            v1 = plsc.load_gather(blk_v.at[base_slot + j], [sv, lane + 16])
            plsc.store_scatter(res_v, [rv, c + lane], v0)
            plsc.store_scatter(res_v, [rv, c + 16 + lane], v1)

    for p in range(NGRP):
        issue(p, p * GS, sems[p])

    def body(t, carry):
        for p in range(NGRP):
            g = NGRP * t + p
            process(g, p * GS, sems[p])
            @pl.when(g + NGRP < NG)
            def _():
                issue(g + NGRP, p * GS, sems[p])
        return carry

    lax.fori_loop(0, NG // NGRP, body, 0)
    pltpu.sync_copy(
        res_v,
        out_hbm.at[pl.ds(pl.multiple_of(wid * (BPW // 4), 8), BPW // 4)])


def _gather_one(idx, tab):
    mesh = plsc.VectorSubcoreMesh(core_axis_name="c", subcore_axis_name="s")
    k = functools.partial(
        pl.kernel,
        mesh=mesh,
        out_type=jax.ShapeDtypeStruct((BATCH // 4, 128), jnp.float32),
        scratch_types=[
            pltpu.VMEM((8, 128), jnp.int32),
            pltpu.VMEM((NGRP * GS, BLK, EMB), jnp.float32),
            pltpu.VMEM((BPW // 4, 128), jnp.float32),
            pltpu.SemaphoreType.DMA,
            pltpu.SemaphoreType.DMA,
            pltpu.SemaphoreType.DMA,
            pltpu.SemaphoreType.DMA,
        ],
        compiler_params=pltpu.CompilerParams(use_tc_tiling_on_sc=True,
                                             needs_layout_passes=False),
    )(_gather_kernel)
    idx8 = jnp.pad(idx.reshape(NW, 4, 128),
                   ((0, 0), (0, 4), (0, 0))).reshape(NW * 8, 128)
    return k(idx8, tab)


def _mlp_kernel(gu_ref, gm_ref, w1u_ref, w1m_ref, b1_ref,
                w2_ref, b2_ref, w3t_ref, b3_ref, out_ref):
    outs = []
    for s in range(4):
        u = gu_ref[:, s * EMB:(s + 1) * EMB]
        m = gm_ref[:, s * EMB:(s + 1) * EMB]
        h = jnp.dot(u, w1u_ref[...], preferred_element_type=jnp.float32)
        h = h + jnp.dot(m, w1m_ref[...], preferred_element_type=jnp.float32)
        h = jnp.maximum(h + b1_ref[...], 0.0)
        h = jnp.dot(h, w2_ref[...], preferred_element_type=jnp.float32)
        h = jnp.maximum(h + b2_ref[...], 0.0)
        o = jnp.sum(h * w3t_ref[...], axis=1, keepdims=True) + b3_ref[...]
        outs.append(1.0 / (1.0 + jnp.exp(-o)))
    out_ref[...] = jnp.concatenate(outs, axis=1)


def _mlp(gu, gm, W1, b1, W2, b2, W3, b3):
    hid = W1.shape[1]
    h2 = W2.shape[1]
    bm4 = BM // 4
    grid = ((BATCH // 4) // bm4,)
    full = lambda shape: pl.BlockSpec(shape, lambda i: (0, 0))
    out = pl.pallas_call(
        _mlp_kernel,
        grid=grid,
        in_specs=[
            pl.BlockSpec((bm4, 128), lambda i: (i, 0)),
            pl.BlockSpec((bm4, 128), lambda i: (i, 0)),
            full((EMB, hid)),
            full((EMB, hid)),
            full((1, hid)),
            full((hid, h2)),
            full((1, h2)),
            full((1, h2)),
            full((1, 1)),
        ],
        out_specs=pl.BlockSpec((bm4, 4), lambda i: (i, 0)),
        out_shape=jax.ShapeDtypeStruct((BATCH // 4, 4), jnp.float32),
    )(gu, gm, W1[:EMB], W1[EMB:], b1.reshape(1, hid), W2,
      b2.reshape(1, h2), W3.reshape(1, h2), b3.reshape(1, 1))
    return out


def kernel(user, movie, user_emb_table, movie_emb_table, W1, b1, W2, b2, W3, b3):
    user = user.astype(jnp.int32)
    movie = movie.astype(jnp.int32)
    gu = _gather_one(user, user_emb_table)
    gm = _gather_one(movie, movie_emb_table)
    out = _mlp(gu, gm, W1, b1, W2, b2, W3, b3)
    return out.reshape(BATCH)
